# Initial kernel scaffold; baseline (speedup 1.0000x reference)
import jax
import jax.numpy as jnp
from jax.experimental import pallas as pl


def kernel(h, e, edge_index, W1, g1, b1, W2, gh, bh):
    raise NotImplementedError("write your pallas kernel here")


# speedup vs baseline: 4.6880x; 4.6880x over previous
import functools

import jax
import jax.numpy as jnp
from jax import lax
from jax.experimental import pallas as pl
from jax.experimental.pallas import tpu as pltpu
from jax.experimental.pallas import tpu_sc as plsc

N = 10000
E = 160000
IND = 256
HID = 512
OUT = 64
H = 4
EPS = 1e-5

NB = 10
BLK = N // NB

N_TILES = 16
CHUNK = 128
NCH = 80
IDXB = 16
E_PER_TILE = NCH * CHUNK
E_PAD = N_TILES * E_PER_TILE
ACC_ROWS = 10240
RPT_ZERO = ACC_ROWS // N_TILES
RPT_OUT = 632
RPT_LAST = N - 15 * RPT_OUT
HALF = 2 * OUT


def _stats_body(h_ref, w1_ref, g1_ref, b1_ref, a_ref, c_ref, c_acc, s_acc):
    b = pl.program_id(0)

    @pl.when(b == 0)
    def _init():
        c_acc[...] = jnp.zeros_like(c_acc)
        s_acc[...] = jnp.zeros_like(s_acc)

    hb = h_ref[...]
    c_acc[...] += lax.dot_general(hb, hb, (((0,), (0,)), ((), ())),
                                  preferred_element_type=jnp.float32)
    s_acc[...] += jnp.sum(hb, axis=0, keepdims=True)

    @pl.when(b == NB - 1)
    def _finish():
        cm = c_acc[...]
        sm = s_acc[...]
        for i in range(H):
            w = w1_ref[i]
            mu = jnp.dot(sm, w, preferred_element_type=jnp.float32) / N
            ex2 = jnp.sum(w * jnp.dot(cm, w, preferred_element_type=jnp.float32),
                          axis=0, keepdims=True) / N
            var = ex2 - mu * mu
            istd = lax.rsqrt(var + EPS)
            g = g1_ref[i]
            a_ref[i] = g * istd
            c_ref[i] = b1_ref[i] - mu * g * istd


def _bn1_stats(h, W1, g1r, b1r):
    return pl.pallas_call(
        _stats_body,
        grid=(NB,),
        in_specs=[
            pl.BlockSpec((BLK, IND), lambda b: (b, 0)),
            pl.BlockSpec((H, IND, HID), lambda b: (0, 0, 0)),
            pl.BlockSpec((H, 1, HID), lambda b: (0, 0, 0)),
            pl.BlockSpec((H, 1, HID), lambda b: (0, 0, 0)),
        ],
        out_specs=[
            pl.BlockSpec((H, 1, HID), lambda b: (0, 0, 0)),
            pl.BlockSpec((H, 1, HID), lambda b: (0, 0, 0)),
        ],
        out_shape=[
            jax.ShapeDtypeStruct((H, 1, HID), jnp.float32),
            jax.ShapeDtypeStruct((H, 1, HID), jnp.float32),
        ],
        scratch_shapes=[
            pltpu.VMEM((IND, IND), jnp.float32),
            pltpu.VMEM((1, IND), jnp.float32),
        ],
    )(h, W1, g1r, b1r)


def _transform_body(h_ref, w1_ref, a_ref, c_ref, w2_ref, zl_ref, zr_ref):
    hb = h_ref[...]
    for i in range(H):
        z1 = jnp.dot(hb, w1_ref[i], preferred_element_type=jnp.float32)
        z1 = jnp.maximum(z1 * a_ref[i] + c_ref[i], 0.0)
        z = jnp.dot(z1, w2_ref[i], preferred_element_type=jnp.float32)
        tgt = zl_ref if i < 2 else zr_ref
        col = (i % 2) * OUT
        tgt[:, col:col + OUT] = z


def _transform(h, W1, a1, c1, W2):
    return pl.pallas_call(
        _transform_body,
        grid=(NB,),
        in_specs=[
            pl.BlockSpec((BLK, IND), lambda b: (b, 0)),
            pl.BlockSpec((H, IND, HID), lambda b: (0, 0, 0)),
            pl.BlockSpec((H, 1, HID), lambda b: (0, 0, 0)),
            pl.BlockSpec((H, 1, HID), lambda b: (0, 0, 0)),
            pl.BlockSpec((H, HID, OUT), lambda b: (0, 0, 0)),
        ],
        out_specs=[
            pl.BlockSpec((BLK, HALF), lambda b: (b, 0)),
            pl.BlockSpec((BLK, HALF), lambda b: (b, 0)),
        ],
        out_shape=[
            jax.ShapeDtypeStruct((N, HALF), jnp.float32),
            jax.ShapeDtypeStruct((N, HALF), jnp.float32),
        ],
    )(h, W1, a1, c1, W2)


def _seg_body(zl_hbm, zr_hbm, src_hbm, dst_hbm, zeros_hbm,
              aggl_hbm, aggr_hbm,
              src_v, dst_v, buf_a, buf_b, acc, sem_a, sem_b):
    c = lax.axis_index("c")
    s = lax.axis_index("s")

    pltpu.sync_copy(zeros_hbm, acc.at[pl.ds(s * RPT_ZERO, RPT_ZERO)])
    plsc.subcore_barrier()

    def run(z_hbm, agg_hbm):
        def stage(st, carry):
            pltpu.sync_copy(src_hbm.at[s, pl.ds(st * IDXB, IDXB)], src_v)
            pltpu.sync_copy(dst_hbm.at[s, pl.ds(st * IDXB, IDXB)], dst_v)
            pltpu.async_copy(z_hbm.at[src_v.at[0]], buf_a, sem_a)

            def body(jj, c2):
                j = 2 * jj
                pltpu.make_async_copy(z_hbm.at[src_v.at[j]], buf_a, sem_a).wait()
                pltpu.async_copy(z_hbm.at[src_v.at[j + 1]], buf_b, sem_b)
                pltpu.sync_copy(buf_a, acc.at[dst_v.at[j]], add=True)
                pltpu.make_async_copy(z_hbm.at[src_v.at[j + 1]], buf_b,
                                      sem_b).wait()

                @pl.when(jj < IDXB // 2 - 1)
                def _fire_next():
                    pltpu.async_copy(z_hbm.at[src_v.at[j + 2]], buf_a, sem_a)

                pltpu.sync_copy(buf_b, acc.at[dst_v.at[j + 1]], add=True)
                return c2

            lax.fori_loop(0, IDXB // 2, body, 0)
            return carry

        lax.fori_loop(0, NCH // IDXB, stage, 0)
        plsc.subcore_barrier()
        @pl.when(s < N_TILES - 1)
        def _most():
            pltpu.sync_copy(acc.at[pl.ds(s * RPT_OUT, RPT_OUT)],
                            agg_hbm.at[pl.ds(s * RPT_OUT, RPT_OUT)])

        @pl.when(s == N_TILES - 1)
        def _last():
            pltpu.sync_copy(acc.at[pl.ds(15 * RPT_OUT, RPT_LAST)],
                            agg_hbm.at[pl.ds(15 * RPT_OUT, RPT_LAST)])

    @pl.when(c == 0)
    def _left():
        run(zl_hbm, aggl_hbm)

    @pl.when(c == 1)
    def _right():
        run(zr_hbm, aggr_hbm)


@functools.cache
def _seg_kernel():
    return pl.kernel(
        _seg_body,
        out_type=[
            jax.ShapeDtypeStruct((N, HALF), jnp.float32),
            jax.ShapeDtypeStruct((N, HALF), jnp.float32),
        ],
        mesh=plsc.VectorSubcoreMesh(core_axis_name="c", subcore_axis_name="s"),
        scratch_types=[
            pltpu.VMEM((IDXB, CHUNK), jnp.int32),
            pltpu.VMEM((IDXB, CHUNK), jnp.int32),
            pltpu.VMEM((CHUNK, HALF), jnp.float32),
            pltpu.VMEM((CHUNK, HALF), jnp.float32),
            pltpu.VMEM_SHARED((ACC_ROWS, HALF), jnp.float32),
            pltpu.SemaphoreType.DMA,
            pltpu.SemaphoreType.DMA,
        ],
    )


def _segment_sum_sc(zl, zr, src_p, dst_p, zeros_blk):
    return _seg_kernel()(zl, zr, src_p, dst_p, zeros_blk)


def _bn2_body(aggl_ref, aggr_ref, h_ref, gh_ref, bh_ref, out_ref,
              s_acc, q_acc, sc_ref, sh_ref):
    p = pl.program_id(0)
    b = pl.program_id(1)

    @pl.when((p == 0) & (b == 0))
    def _init():
        s_acc[...] = jnp.zeros_like(s_acc)
        q_acc[...] = jnp.zeros_like(q_acc)

    agg = jnp.concatenate([aggl_ref[...], aggr_ref[...]], axis=1)

    @pl.when(p == 0)
    def _accum():
        s_acc[...] += jnp.sum(agg, axis=0, keepdims=True)
        q_acc[...] += jnp.sum(agg * agg, axis=0, keepdims=True)

    @pl.when((p == 0) & (b == NB - 1))
    def _finish():
        mu = s_acc[...] / N
        var = q_acc[...] / N - mu * mu
        istd = lax.rsqrt(var + EPS)
        sc_ref[...] = gh_ref[...] * istd
        sh_ref[...] = bh_ref[...] - mu * gh_ref[...] * istd

    @pl.when(p == 1)
    def _write():
        out_ref[...] = h_ref[...] + jnp.maximum(agg * sc_ref[...] + sh_ref[...],
                                                0.0)


def _bn2_residual(aggl, aggr, h, ghr, bhr):
    return pl.pallas_call(
        _bn2_body,
        grid=(2, NB),
        in_specs=[
            pl.BlockSpec((BLK, HALF), lambda p, b: (b, 0)),
            pl.BlockSpec((BLK, HALF), lambda p, b: (b, 0)),
            pl.BlockSpec((BLK, IND), lambda p, b: (b, 0)),
            pl.BlockSpec((1, IND), lambda p, b: (0, 0)),
            pl.BlockSpec((1, IND), lambda p, b: (0, 0)),
        ],
        out_specs=pl.BlockSpec((BLK, IND), lambda p, b: (b, 0)),
        out_shape=jax.ShapeDtypeStruct((N, IND), jnp.float32),
        scratch_shapes=[
            pltpu.VMEM((1, IND), jnp.float32),
            pltpu.VMEM((1, IND), jnp.float32),
            pltpu.VMEM((1, IND), jnp.float32),
            pltpu.VMEM((1, IND), jnp.float32),
        ],
    )(aggl, aggr, h, ghr, bhr)


def kernel(h, e, edge_index, W1, g1, b1, W2, gh, bh):
    src = edge_index[0]
    dst = edge_index[1]
    pad = E_PAD - E
    src_p = jnp.concatenate([src, jnp.zeros((pad,), jnp.int32)])
    src_p = src_p.reshape(N_TILES, NCH, CHUNK)
    dst_p = jnp.concatenate([dst, jnp.full((pad,), N, jnp.int32)])
    dst_p = dst_p.reshape(N_TILES, NCH, CHUNK)
    zeros_blk = jnp.zeros((RPT_ZERO, HALF), jnp.float32)

    g1r = g1.reshape(H, 1, HID)
    b1r = b1.reshape(H, 1, HID)
    ghr = gh.reshape(1, H * OUT)
    bhr = bh.reshape(1, H * OUT)

    a1, c1 = _bn1_stats(h, W1, g1r, b1r)
    zl, zr = _transform(h, W1, a1, c1, W2)
    aggl, aggr = _segment_sum_sc(zl, zr, src_p, dst_p, zeros_blk)
    out = _bn2_residual(aggl, aggr, h, ghr, bhr)
    return (out, e)


# speedup vs baseline: 4.6947x; 1.0014x over previous
import functools

import jax
import jax.numpy as jnp
from jax import lax
from jax.experimental import pallas as pl
from jax.experimental.pallas import tpu as pltpu
from jax.experimental.pallas import tpu_sc as plsc

N = 10000
E = 160000
IND = 256
HID = 512
OUT = 64
H = 4
EPS = 1e-5

NB = 10
BLK = N // NB

N_TILES = 16
CHUNK = 128
NCH = 80
IDXB = 16
E_PER_TILE = NCH * CHUNK
E_PAD = N_TILES * E_PER_TILE
ACC_ROWS = 10240
RPT_ZERO = ACC_ROWS // N_TILES
RPT_OUT = 632
RPT_LAST = N - 15 * RPT_OUT
HALF = 2 * OUT


def _stats_body(h_ref, w1_ref, g1_ref, b1_ref, a_ref, c_ref, c_acc, s_acc):
    b = pl.program_id(0)

    @pl.when(b == 0)
    def _init():
        c_acc[...] = jnp.zeros_like(c_acc)
        s_acc[...] = jnp.zeros_like(s_acc)

    hb = h_ref[...]
    c_acc[...] += lax.dot_general(hb, hb, (((0,), (0,)), ((), ())),
                                  preferred_element_type=jnp.float32)
    s_acc[...] += jnp.sum(hb, axis=0, keepdims=True)

    @pl.when(b == NB - 1)
    def _finish():
        cm = c_acc[...]
        sm = s_acc[...]
        for i in range(H):
            w = w1_ref[i]
            mu = jnp.dot(sm, w, preferred_element_type=jnp.float32) / N
            ex2 = jnp.sum(w * jnp.dot(cm, w, preferred_element_type=jnp.float32),
                          axis=0, keepdims=True) / N
            var = ex2 - mu * mu
            istd = lax.rsqrt(var + EPS)
            g = g1_ref[i]
            a_ref[i] = g * istd
            c_ref[i] = b1_ref[i] - mu * g * istd


def _bn1_stats(h, W1, g1r, b1r):
    return pl.pallas_call(
        _stats_body,
        grid=(NB,),
        in_specs=[
            pl.BlockSpec((BLK, IND), lambda b: (b, 0)),
            pl.BlockSpec((H, IND, HID), lambda b: (0, 0, 0)),
            pl.BlockSpec((H, 1, HID), lambda b: (0, 0, 0)),
            pl.BlockSpec((H, 1, HID), lambda b: (0, 0, 0)),
        ],
        out_specs=[
            pl.BlockSpec((H, 1, HID), lambda b: (0, 0, 0)),
            pl.BlockSpec((H, 1, HID), lambda b: (0, 0, 0)),
        ],
        out_shape=[
            jax.ShapeDtypeStruct((H, 1, HID), jnp.float32),
            jax.ShapeDtypeStruct((H, 1, HID), jnp.float32),
        ],
        scratch_shapes=[
            pltpu.VMEM((IND, IND), jnp.float32),
            pltpu.VMEM((1, IND), jnp.float32),
        ],
    )(h, W1, g1r, b1r)


def _transform_body(h_ref, w1_ref, a_ref, c_ref, w2_ref, zl_ref, zr_ref):
    hb = h_ref[...]
    for i in range(H):
        z1 = jnp.dot(hb, w1_ref[i], preferred_element_type=jnp.float32)
        z1 = jnp.maximum(z1 * a_ref[i] + c_ref[i], 0.0)
        z = jnp.dot(z1.astype(jnp.bfloat16), w2_ref[i],
                    preferred_element_type=jnp.float32)
        tgt = zl_ref if i < 2 else zr_ref
        col = (i % 2) * OUT
        tgt[:, col:col + OUT] = z


def _transform(h, W1, a1, c1, W2):
    return pl.pallas_call(
        _transform_body,
        grid=(NB,),
        in_specs=[
            pl.BlockSpec((BLK, IND), lambda b: (b, 0)),
            pl.BlockSpec((H, IND, HID), lambda b: (0, 0, 0)),
            pl.BlockSpec((H, 1, HID), lambda b: (0, 0, 0)),
            pl.BlockSpec((H, 1, HID), lambda b: (0, 0, 0)),
            pl.BlockSpec((H, HID, OUT), lambda b: (0, 0, 0)),
        ],
        out_specs=[
            pl.BlockSpec((BLK, HALF), lambda b: (b, 0)),
            pl.BlockSpec((BLK, HALF), lambda b: (b, 0)),
        ],
        out_shape=[
            jax.ShapeDtypeStruct((N, HALF), jnp.float32),
            jax.ShapeDtypeStruct((N, HALF), jnp.float32),
        ],
    )(h, W1, a1, c1, W2)


def _seg_body(zl_hbm, zr_hbm, src_hbm, dst_hbm, zeros_hbm,
              aggl_hbm, aggr_hbm,
              src_v, dst_v, buf_a, buf_b, acc, sem_a, sem_b):
    c = lax.axis_index("c")
    s = lax.axis_index("s")

    pltpu.sync_copy(zeros_hbm, acc.at[pl.ds(s * RPT_ZERO, RPT_ZERO)])
    plsc.subcore_barrier()

    def run(z_hbm, agg_hbm):
        def stage(st, carry):
            pltpu.sync_copy(src_hbm.at[s, pl.ds(st * IDXB, IDXB)], src_v)
            pltpu.sync_copy(dst_hbm.at[s, pl.ds(st * IDXB, IDXB)], dst_v)
            pltpu.async_copy(z_hbm.at[src_v.at[0]], buf_a, sem_a)

            def body(jj, c2):
                j = 2 * jj
                pltpu.make_async_copy(z_hbm.at[src_v.at[j]], buf_a, sem_a).wait()
                pltpu.async_copy(z_hbm.at[src_v.at[j + 1]], buf_b, sem_b)
                pltpu.sync_copy(buf_a, acc.at[dst_v.at[j]], add=True)
                pltpu.make_async_copy(z_hbm.at[src_v.at[j + 1]], buf_b,
                                      sem_b).wait()

                @pl.when(jj < IDXB // 2 - 1)
                def _fire_next():
                    pltpu.async_copy(z_hbm.at[src_v.at[j + 2]], buf_a, sem_a)

                pltpu.sync_copy(buf_b, acc.at[dst_v.at[j + 1]], add=True)
                return c2

            lax.fori_loop(0, IDXB // 2, body, 0)
            return carry

        lax.fori_loop(0, NCH // IDXB, stage, 0)
        plsc.subcore_barrier()
        @pl.when(s < N_TILES - 1)
        def _most():
            pltpu.sync_copy(acc.at[pl.ds(s * RPT_OUT, RPT_OUT)],
                            agg_hbm.at[pl.ds(s * RPT_OUT, RPT_OUT)])

        @pl.when(s == N_TILES - 1)
        def _last():
            pltpu.sync_copy(acc.at[pl.ds(15 * RPT_OUT, RPT_LAST)],
                            agg_hbm.at[pl.ds(15 * RPT_OUT, RPT_LAST)])

    @pl.when(c == 0)
    def _left():
        run(zl_hbm, aggl_hbm)

    @pl.when(c == 1)
    def _right():
        run(zr_hbm, aggr_hbm)


@functools.cache
def _seg_kernel():
    return pl.kernel(
        _seg_body,
        out_type=[
            jax.ShapeDtypeStruct((N, HALF), jnp.float32),
            jax.ShapeDtypeStruct((N, HALF), jnp.float32),
        ],
        mesh=plsc.VectorSubcoreMesh(core_axis_name="c", subcore_axis_name="s"),
        scratch_types=[
            pltpu.VMEM((IDXB, CHUNK), jnp.int32),
            pltpu.VMEM((IDXB, CHUNK), jnp.int32),
            pltpu.VMEM((CHUNK, HALF), jnp.float32),
            pltpu.VMEM((CHUNK, HALF), jnp.float32),
            pltpu.VMEM_SHARED((ACC_ROWS, HALF), jnp.float32),
            pltpu.SemaphoreType.DMA,
            pltpu.SemaphoreType.DMA,
        ],
    )


def _segment_sum_sc(zl, zr, src_p, dst_p, zeros_blk):
    return _seg_kernel()(zl, zr, src_p, dst_p, zeros_blk)


def _bn2_body(aggl_ref, aggr_ref, h_ref, gh_ref, bh_ref, out_ref,
              s_acc, q_acc, sc_ref, sh_ref):
    p = pl.program_id(0)
    b = pl.program_id(1)

    @pl.when((p == 0) & (b == 0))
    def _init():
        s_acc[...] = jnp.zeros_like(s_acc)
        q_acc[...] = jnp.zeros_like(q_acc)

    agg = jnp.concatenate([aggl_ref[...], aggr_ref[...]], axis=1)

    @pl.when(p == 0)
    def _accum():
        s_acc[...] += jnp.sum(agg, axis=0, keepdims=True)
        q_acc[...] += jnp.sum(agg * agg, axis=0, keepdims=True)

    @pl.when((p == 0) & (b == NB - 1))
    def _finish():
        mu = s_acc[...] / N
        var = q_acc[...] / N - mu * mu
        istd = lax.rsqrt(var + EPS)
        sc_ref[...] = gh_ref[...] * istd
        sh_ref[...] = bh_ref[...] - mu * gh_ref[...] * istd

    @pl.when(p == 1)
    def _write():
        out_ref[...] = h_ref[...] + jnp.maximum(agg * sc_ref[...] + sh_ref[...],
                                                0.0)


def _bn2_residual(aggl, aggr, h, ghr, bhr):
    return pl.pallas_call(
        _bn2_body,
        grid=(2, NB),
        in_specs=[
            pl.BlockSpec((BLK, HALF), lambda p, b: (b, 0)),
            pl.BlockSpec((BLK, HALF), lambda p, b: (b, 0)),
            pl.BlockSpec((BLK, IND), lambda p, b: (b, 0)),
            pl.BlockSpec((1, IND), lambda p, b: (0, 0)),
            pl.BlockSpec((1, IND), lambda p, b: (0, 0)),
        ],
        out_specs=pl.BlockSpec((BLK, IND), lambda p, b: (b, 0)),
        out_shape=jax.ShapeDtypeStruct((N, IND), jnp.float32),
        scratch_shapes=[
            pltpu.VMEM((1, IND), jnp.float32),
            pltpu.VMEM((1, IND), jnp.float32),
            pltpu.VMEM((1, IND), jnp.float32),
            pltpu.VMEM((1, IND), jnp.float32),
        ],
    )(aggl, aggr, h, ghr, bhr)


def kernel(h, e, edge_index, W1, g1, b1, W2, gh, bh):
    src = edge_index[0]
    dst = edge_index[1]
    pad = E_PAD - E
    src_p = jnp.concatenate([src, jnp.zeros((pad,), jnp.int32)])
    src_p = src_p.reshape(N_TILES, NCH, CHUNK)
    dst_p = jnp.concatenate([dst, jnp.full((pad,), N, jnp.int32)])
    dst_p = dst_p.reshape(N_TILES, NCH, CHUNK)
    zeros_blk = jnp.zeros((RPT_ZERO, HALF), jnp.float32)

    g1r = g1.reshape(H, 1, HID)
    b1r = b1.reshape(H, 1, HID)
    ghr = gh.reshape(1, H * OUT)
    bhr = bh.reshape(1, H * OUT)

    a1, c1 = _bn1_stats(h, W1, g1r, b1r)
    zl, zr = _transform(h.astype(jnp.bfloat16), W1.astype(jnp.bfloat16),
                        a1, c1, W2.astype(jnp.bfloat16))
    aggl, aggr = _segment_sum_sc(zl, zr, src_p, dst_p, zeros_blk)
    out = _bn2_residual(aggl, aggr, h, ghr, bhr)
    return (out, e)
